# Initial kernel scaffold; baseline (speedup 1.0000x reference)
import jax
import jax.numpy as jnp
from jax.experimental import pallas as pl


def kernel(obs_vec, W_root, b_root, Wx1, bx1, Wx2, bx2, Wy1, by1, Wy2, by2, Wz1, bz1, Wz2, bz2, Wz3, bz3):
    raise NotImplementedError("write your pallas kernel here")


# speedup vs baseline: 1.0221x; 1.0221x over previous
import functools

import jax
import jax.numpy as jnp
from jax import lax
from jax.experimental import pallas as pl
from jax.experimental.pallas import tpu as pltpu
from jax.experimental.pallas import tpu_sc as plsc

B = 16384
N_MODES = 16
TYPES = 3
D_OBS = 10
D_H = 32
D_E = 8
D_Z1 = TYPES * D_H
D_A = D_Z1 + D_H
D_Z2 = TYPES * 16
D_Z3 = TYPES * N_MODES

NC = 2
NS = 16
L = 16
NW = NC * NS
TPW = B // NW
VPW = TPW // L

BT = 4096


def _tc_body(obs_ref, wa_ref, ba_ref, w2_ref, b2_ref, w3_ref, b3_ref,
             zall_ref, tid_ref, vs_ref):
    obs = obs_ref[...]
    a = jnp.maximum(
        jnp.dot(obs, wa_ref[...], preferred_element_type=jnp.float32)
        + ba_ref[...], 0.0)
    h2 = jnp.maximum(
        jnp.dot(a, w2_ref[...], preferred_element_type=jnp.float32)
        + b2_ref[...], 0.0)
    zall_ref[...] = (jnp.dot(h2, w3_ref[...],
                             preferred_element_type=jnp.float32)
                     + b3_ref[...])
    tid_ref[...] = obs[:, 8:9].astype(jnp.int32)
    vs_ref[...] = a[:, D_Z1:D_A]


def _tc_stage(obs, wa, ba, w2, b2, w3, b3):
    rep = lambda shape: pl.BlockSpec(shape, lambda i: (0, 0))
    return pl.pallas_call(
        _tc_body,
        grid=(B // BT,),
        in_specs=[
            pl.BlockSpec((BT, D_OBS), lambda i: (i, 0)),
            rep((D_OBS, D_A)), rep((1, D_A)),
            rep((D_A, D_Z2)), rep((1, D_Z2)),
            rep((D_Z2, D_Z3)), rep((1, D_Z3)),
        ],
        out_specs=[
            pl.BlockSpec((BT, D_Z3), lambda i: (i, 0)),
            pl.BlockSpec((BT, 1), lambda i: (i, 0)),
            pl.BlockSpec((BT, D_H), lambda i: (i, 0)),
        ],
        out_shape=[
            jax.ShapeDtypeStruct((B, D_Z3), jnp.float32),
            jax.ShapeDtypeStruct((B, 1), jnp.int32),
            jax.ShapeDtypeStruct((B, D_H), jnp.float32),
        ],
    )(obs, wa, ba, w2, b2, w3, b3)


def _sc_body(zall_hbm, tid_hbm, vs_hbm, wx1_h, bx1_h, wx2_h, bx2_h,
             wy1_h, by1_h, wy2_h, by2_h, zout_hbm, act_hbm,
             zall_v, tid_v, vs_v, wx1_v, bx1_v, wx2_v, bx2_v,
             wy1_v, by1_v, wy2_v, by2_v, zout_v, act_v):
    wid = lax.axis_index("s") * NC + lax.axis_index("c")
    base = wid * TPW
    pltpu.sync_copy(zall_hbm.at[pl.ds(base * D_Z3, TPW * D_Z3)], zall_v)
    pltpu.sync_copy(tid_hbm.at[pl.ds(base, TPW)], tid_v)
    pltpu.sync_copy(vs_hbm.at[pl.ds(base * D_H, TPW * D_H)], vs_v)
    pltpu.sync_copy(wx1_h, wx1_v)
    pltpu.sync_copy(bx1_h, bx1_v)
    pltpu.sync_copy(wx2_h, wx2_v)
    pltpu.sync_copy(bx2_h, bx2_v)
    pltpu.sync_copy(wy1_h, wy1_v)
    pltpu.sync_copy(by1_h, by1_v)
    pltpu.sync_copy(wy2_h, wy2_v)
    pltpu.sync_copy(by2_h, by2_v)

    lane = lax.iota(jnp.int32, L)
    neg_inf = jnp.full((L,), -jnp.inf, jnp.float32)

    def per_vreg(v, c):
        tok = lane + v * L
        tid = tid_v[pl.ds(v * L, L)]
        zb = tok * D_Z3 + tid * N_MODES
        zo = tok * N_MODES
        best_val = neg_inf
        best = jnp.zeros((L,), jnp.int32)
        for m in range(N_MODES):
            zm = plsc.load_gather(zall_v, [zb + m])
            plsc.store_scatter(zout_v, [zo + m], zm)
            gt = zm > best_val
            best_val = jnp.where(gt, zm, best_val)
            best = jnp.where(gt, m, best)

        mb1 = best * (D_H * D_E)
        mbb = best * D_E
        hx = tuple(plsc.load_gather(bx1_v, [mbb + k]) for k in range(D_E))
        hy = tuple(plsc.load_gather(by1_v, [mbb + k]) for k in range(D_E))
        vsbase = tok * D_H

        def dstep(d, carry):
            hx, hy = carry
            vsd = plsc.load_gather(vs_v, [vsbase + d])
            wb = mb1 + d * D_E
            hx = tuple(hx[k] + vsd * plsc.load_gather(wx1_v, [wb + k])
                       for k in range(D_E))
            hy = tuple(hy[k] + vsd * plsc.load_gather(wy1_v, [wb + k])
                       for k in range(D_E))
            return hx, hy

        hx, hy = lax.fori_loop(0, D_H, dstep, (hx, hy))
        lx = plsc.load_gather(bx2_v, [best])
        ly = plsc.load_gather(by2_v, [best])
        for k in range(D_E):
            lx = lx + jnp.maximum(hx[k], 0.0) * plsc.load_gather(wx2_v, [mbb + k])
            ly = ly + jnp.maximum(hy[k], 0.0) * plsc.load_gather(wy2_v, [mbb + k])
        plsc.store_scatter(act_v, [tok * 2], lx)
        plsc.store_scatter(act_v, [tok * 2 + 1], ly)
        return c

    lax.fori_loop(0, VPW, per_vreg, 0)
    pltpu.sync_copy(zout_v, zout_hbm.at[pl.ds(base * N_MODES, TPW * N_MODES)])
    pltpu.sync_copy(act_v, act_hbm.at[pl.ds(base * 2, TPW * 2)])


_SC_SCRATCH = [
    pltpu.VMEM((TPW * D_Z3,), jnp.float32),
    pltpu.VMEM((TPW,), jnp.int32),
    pltpu.VMEM((TPW * D_H,), jnp.float32),
    pltpu.VMEM((N_MODES * D_H * D_E,), jnp.float32),
    pltpu.VMEM((N_MODES * D_E,), jnp.float32),
    pltpu.VMEM((N_MODES * D_E,), jnp.float32),
    pltpu.VMEM((N_MODES,), jnp.float32),
    pltpu.VMEM((N_MODES * D_H * D_E,), jnp.float32),
    pltpu.VMEM((N_MODES * D_E,), jnp.float32),
    pltpu.VMEM((N_MODES * D_E,), jnp.float32),
    pltpu.VMEM((N_MODES,), jnp.float32),
    pltpu.VMEM((TPW * N_MODES,), jnp.float32),
    pltpu.VMEM((TPW * 2,), jnp.float32),
]


@functools.cache
def _sc_stage_built():
    return functools.partial(
        pl.kernel,
        out_type=[
            jax.ShapeDtypeStruct((B * N_MODES,), jnp.float32),
            jax.ShapeDtypeStruct((B * 2,), jnp.float32),
        ],
        mesh=plsc.VectorSubcoreMesh(core_axis_name="c", subcore_axis_name="s",
                                    num_cores=NC, num_subcores=NS),
        scratch_types=_SC_SCRATCH,
        compiler_params=pltpu.CompilerParams(needs_layout_passes=False),
    )(_sc_body)


def kernel(obs_vec, W_root, b_root, Wx1, bx1, Wx2, bx2, Wy1, by1, Wy2, by2,
           Wz1, bz1, Wz2, bz2, Wz3, bz3):
    wz1c = jnp.transpose(Wz1, (1, 0, 2)).reshape(D_OBS, D_Z1)
    wa = jnp.concatenate([wz1c, W_root], axis=1)
    ba = jnp.concatenate([bz1.reshape(-1), b_root]).reshape(1, D_A)
    w2 = jnp.concatenate(
        [jax.scipy.linalg.block_diag(*[Wz2[t] for t in range(TYPES)]),
         jnp.zeros((D_H, D_Z2), jnp.float32)], axis=0)
    b2 = bz2.reshape(1, D_Z2)
    w3 = jax.scipy.linalg.block_diag(*[Wz3[t] for t in range(TYPES)])
    b3 = bz3.reshape(1, D_Z3)

    z_all, tid, vs = _tc_stage(obs_vec, wa, ba, w2, b2, w3, b3)

    z_flat, act_flat = _sc_stage_built()(
        z_all.reshape(-1), tid.reshape(-1), vs.reshape(-1),
        Wx1.reshape(-1), bx1.reshape(-1),
        Wx2[:, :, 0].reshape(-1), bx2[:, 0],
        Wy1.reshape(-1), by1.reshape(-1),
        Wy2[:, :, 0].reshape(-1), by2[:, 0],
    )
    return act_flat.reshape(B, 2), z_flat.reshape(B, N_MODES)
